# baseline (speedup 1.0000x reference)
import functools
import math

import jax
import jax.numpy as jnp
from jax import lax
from jax.experimental import pallas as pl
from jax.experimental.pallas import tpu as pltpu
from jax.experimental.pallas import tpu_sc as plsc

SCALE = 32.0
ALPHA = 0.2
CBLK = 2048
NEG = -1e30


def _gather_targets(y32, score_flat, batch, num_cls):
    info = plsc.get_sparse_core_info()
    nw = info.num_cores * info.num_subcores
    bpw = batch // nw

    mesh = plsc.VectorSubcoreMesh(core_axis_name="c", subcore_axis_name="s")

    @functools.partial(
        pl.kernel,
        mesh=mesh,
        out_type=jax.ShapeDtypeStruct((batch,), jnp.float32),
        scratch_types=[
            pltpu.VMEM((bpw,), jnp.int32),
            pltpu.VMEM((bpw,), jnp.int32),
            pltpu.VMEM((bpw,), jnp.float32),
            pltpu.SemaphoreType.DMA,
        ],
    )
    def k(y_hbm, flat_hbm, out_hbm, y_v, idx_v, vals_v, sem):
        wid = lax.axis_index("s") * info.num_cores + lax.axis_index("c")
        base = wid * bpw
        pltpu.sync_copy(y_hbm.at[pl.ds(base, bpw)], y_v)
        for c in range(bpw // 16):
            rows = base + c * 16 + lax.iota(jnp.int32, 16)
            idx_v[pl.ds(c * 16, 16)] = y_v[pl.ds(c * 16, 16)] + rows * num_cls
        pltpu.async_copy(flat_hbm.at[idx_v], vals_v, sem).wait()
        pltpu.sync_copy(vals_v, out_hbm.at[pl.ds(base, bpw)])

    return k(y32, score_flat)


def _dense_loss(score, t_col, batch, num_cls):
    ncb = pl.cdiv(num_cls, CBLK)
    corr = math.exp(-SCALE * ALPHA) - 1.0

    def body(t_ref, score_ref, out_ref, m_ref, s_ref):
        j = pl.program_id(0)

        @pl.when(j == 0)
        def _init():
            m_ref[...] = jnp.full_like(m_ref, NEG)
            s_ref[...] = jnp.zeros_like(s_ref)

        cols = j * CBLK + lax.broadcasted_iota(jnp.int32, (batch, CBLK), 1)
        v = jnp.where(cols < num_cls, score_ref[...] * SCALE, NEG)
        m_old = m_ref[...]
        m_new = jnp.maximum(m_old, jnp.max(v, axis=1, keepdims=True))
        s_ref[...] = s_ref[...] * jnp.exp(m_old - m_new) + jnp.sum(
            jnp.exp(v - m_new), axis=1, keepdims=True
        )
        m_ref[...] = m_new

        @pl.when(j == ncb - 1)
        def _finish():
            tt = t_ref[...] * SCALE
            m = m_ref[...]
            s = s_ref[...] + jnp.exp(tt - m) * corr
            out_ref[...] = m + jnp.log(s) - tt + SCALE * ALPHA

    return pl.pallas_call(
        body,
        grid=(ncb,),
        in_specs=[
            pl.BlockSpec((batch, 1), lambda j: (0, 0)),
            pl.BlockSpec((batch, CBLK), lambda j: (0, j)),
        ],
        out_specs=pl.BlockSpec((batch, 1), lambda j: (0, 0)),
        out_shape=jax.ShapeDtypeStruct((batch, 1), jnp.float32),
        scratch_shapes=[
            pltpu.VMEM((batch, 1), jnp.float32),
            pltpu.VMEM((batch, 1), jnp.float32),
        ],
        compiler_params=pltpu.CompilerParams(
            dimension_semantics=("arbitrary",)
        ),
    )(t_col, score)


def kernel(score, y):
    batch, num_cls = score.shape
    y32 = jnp.asarray(y).reshape(-1).astype(jnp.int32)
    t = _gather_targets(y32, score.reshape(-1), batch, num_cls)
    out = _dense_loss(score, t.reshape(batch, 1), batch, num_cls)
    return out[:, 0]
